# baseline (speedup 1.0000x reference)
import functools

import jax
import jax.numpy as jnp
from jax import lax
from jax.experimental import pallas as pl
from jax.experimental.pallas import tpu as pltpu
from jax.experimental.pallas import tpu_sc as plsc

BATCH = 16384
EMBED_DIM = 32
LANES = 16
NUM_WORKERS = 32
B_PER_W = BATCH // NUM_WORKERS
IDX_CHUNK = 128
GROUPS = B_PER_W // LANES


def _sc_kernel(users_hbm, product_hbm, wbb_hbm, utable_hbm, ptable_hbm,
               out_hbm, idx_u, idx_p, urows, prows, wbb_v, out_v, sem):
    nc = 2
    wid = lax.axis_index("s") * nc + lax.axis_index("c")
    base = wid * B_PER_W

    pltpu.sync_copy(users_hbm.at[pl.ds(base, B_PER_W)], idx_u)
    pltpu.sync_copy(product_hbm.at[pl.ds(base, B_PER_W)], idx_p)
    pltpu.sync_copy(wbb_hbm, wbb_v)

    copies = []
    for c in range(B_PER_W // IDX_CHUNK):
        sl = pl.ds(c * IDX_CHUNK, IDX_CHUNK)
        copies.append(pltpu.async_copy(
            utable_hbm.at[idx_u.at[sl]], urows.at[sl], sem))
        copies.append(pltpu.async_copy(
            ptable_hbm.at[idx_p.at[sl]], prows.at[sl], sem))
    for cp in copies:
        cp.wait()

    def group_body(g, _):
        row_idx = g * LANES + lax.iota(jnp.int32, LANES)
        acc = wbb_v[2 * EMBED_DIM]
        for d in range(EMBED_DIM):
            col = jnp.full((LANES,), d, jnp.int32)
            acc = acc + plsc.load_gather(urows, [row_idx, col]) * wbb_v[d]
            acc = acc + plsc.load_gather(prows, [row_idx, col]) * wbb_v[EMBED_DIM + d]
        out_v[pl.ds(g * LANES, LANES)] = acc
        return ()

    lax.fori_loop(0, GROUPS, group_body, (), unroll=False)

    pltpu.sync_copy(out_v, out_hbm.at[pl.ds(base, B_PER_W)])


@jax.jit
def _run(users, product, wbb, user_table, product_table):
    mesh = plsc.VectorSubcoreMesh(core_axis_name="c", subcore_axis_name="s")
    f = functools.partial(
        pl.kernel,
        out_type=jax.ShapeDtypeStruct((BATCH,), jnp.float32),
        mesh=mesh,
        compiler_params=pltpu.CompilerParams(
            needs_layout_passes=False, use_tc_tiling_on_sc=False),
        scratch_types=[
            pltpu.VMEM((B_PER_W,), jnp.int32),
            pltpu.VMEM((B_PER_W,), jnp.int32),
            pltpu.VMEM((B_PER_W, EMBED_DIM), jnp.float32),
            pltpu.VMEM((B_PER_W, EMBED_DIM), jnp.float32),
            pltpu.VMEM((2 * EMBED_DIM + 1, LANES), jnp.float32),
            pltpu.VMEM((B_PER_W,), jnp.float32),
            pltpu.SemaphoreType.DMA,
        ],
    )(_sc_kernel)
    return f(users, product, wbb, user_table, product_table)


def kernel(users, product, user_table, product_table, W, b):
    wb = jnp.concatenate([W[0], b])
    wbb = jnp.broadcast_to(wb[:, None], (2 * EMBED_DIM + 1, LANES))
    out = _run(users.astype(jnp.int32), product.astype(jnp.int32),
               wbb.astype(jnp.float32), user_table, product_table)
    return out.reshape(BATCH, 1)
